# Initial kernel scaffold; baseline (speedup 1.0000x reference)
import jax
import jax.numpy as jnp
from jax.experimental import pallas as pl


def kernel(x, edge_index, W, a_src, a_tgt, bias):
    raise NotImplementedError("write your pallas kernel here")


# speedup vs baseline: 56.2995x; 56.2995x over previous
import functools

import jax
import jax.numpy as jnp
from jax import lax
from jax.experimental import pallas as pl
from jax.experimental.pallas import tpu as pltpu
from jax.experimental.pallas import tpu_sc as plsc

N = 10000
E = 320000
DIN = 128
H = 8
F = 16
HF = H * F

NC = 2
NS = 16
NW = NC * NS
CHUNK = 128
CPT = (E + NW * CHUNK - 1) // (NW * CHUNK)
EPT = CPT * CHUNK
E_PAD = NW * EPT
NPAD = 10240
ZR = NPAD // NS


def _k1_body(x_ref, w_ref, as_ref, at_ref, proj_ref, ss_ref, st_ref, c_ref):
    proj = jnp.dot(x_ref[...], w_ref[...], preferred_element_type=jnp.float32)
    proj_ref[...] = proj
    ss = jnp.dot(proj, as_ref[...], preferred_element_type=jnp.float32)
    st = jnp.dot(proj, at_ref[...], preferred_element_type=jnp.float32)
    ss_ref[...] = ss
    st_ref[...] = st
    z = jnp.max(ss, axis=0, keepdims=True) + jnp.max(st, axis=0, keepdims=True)
    c_ref[...] = jnp.maximum(z, 0.2 * z)


def _k2_body(proj_h, ss_h, st_h, c_h, pk_h, acc_o, den_o,
             pk_c, src_c, dst_c, ssg, stg, projg, p2d, cvec, zbuf, zbuf16,
             acc_s, den_s, sem):
    c = lax.axis_index("c")
    s = lax.axis_index("s")
    w = c * NS + s

    @pl.loop(0, 16)
    def _(i):
        for j in range(HF // 16):
            zbuf[i, pl.ds(j * 16, 16)] = jnp.zeros((16,), jnp.float32)
        zbuf16[i, :] = jnp.zeros((16,), jnp.float32)

    @pl.loop(0, ZR // 16)
    def _(k):
        pltpu.sync_copy(zbuf, acc_s.at[pl.ds(s * ZR + k * 16, 16)])
        pltpu.sync_copy(zbuf16, den_s.at[pl.ds(s * ZR + k * 16, 16)])

    pltpu.sync_copy(c_h, cvec)
    plsc.subcore_barrier()

    @pl.loop(0, CPT)
    def _(j):
        pltpu.sync_copy(pk_h.at[w, j], pk_c)
        for k in range(CHUNK // 16):
            v = pk_c[pl.ds(k * 16, 16)]
            src_c[pl.ds(k * 16, 16)] = jnp.bitwise_and(v, 16383)
            dst_c[pl.ds(k * 16, 16)] = jnp.right_shift(v, 14)

        pltpu.sync_copy(ss_h.at[src_c], ssg)
        pltpu.sync_copy(st_h.at[dst_c], stg)
        pltpu.sync_copy(proj_h.at[src_c], projg)
        cv = cvec[...]

        @pl.loop(0, CHUNK)
        def _(e):
            sc = ssg[e, :] + stg[e, :]
            sc = jnp.maximum(sc, 0.2 * sc)
            p2d[e, :] = jnp.exp(sc - cv)

        @pl.loop(0, CHUNK)
        def _(e):
            pv = p2d[e, :]
            for h in range(H):
                ph = pv[h]
                projg[e, pl.ds(h * 16, 16)] = projg[e, pl.ds(h * 16, 16)] * ph

        pltpu.sync_copy(p2d, den_s.at[dst_c], add=True)
        pltpu.sync_copy(projg, acc_s.at[dst_c], add=True)

    plsc.subcore_barrier()

    @pl.when(s < NS - 1)
    def _():
        pltpu.sync_copy(acc_s.at[pl.ds(s * ZR, ZR)], acc_o.at[c, pl.ds(s * ZR, ZR)])
        pltpu.sync_copy(den_s.at[pl.ds(s * ZR, ZR)], den_o.at[c, pl.ds(s * ZR, ZR)])

    @pl.when(s == NS - 1)
    def _():
        last = N - (NS - 1) * ZR
        pltpu.sync_copy(acc_s.at[pl.ds((NS - 1) * ZR, last)],
                        acc_o.at[c, pl.ds((NS - 1) * ZR, last)])
        pltpu.sync_copy(den_s.at[pl.ds((NS - 1) * ZR, last)],
                        den_o.at[c, pl.ds((NS - 1) * ZR, last)])


def _k3_body(acc_ref, den_ref, b_ref, bias_ref, out_ref):
    den = den_ref[0] + den_ref[1] + 1e-16
    r = jnp.dot(1.0 / den, b_ref[...], preferred_element_type=jnp.float32)
    v = (acc_ref[0] + acc_ref[1]) * r + bias_ref[...]
    out_ref[...] = jnp.where(v > 0, v, jnp.exp(v) - 1.0)


def kernel(x, edge_index, W, a_src, a_tgt, bias):
    f32 = jnp.float32

    a_s = a_src.reshape(HF)
    a_t = a_tgt.reshape(HF)
    sel = (jnp.arange(HF)[:, None] // F == jnp.arange(H)[None, :]).astype(f32)
    A_src = jnp.tile(sel * a_s[:, None], (1, 2))
    A_tgt = jnp.tile(sel * a_t[:, None], (1, 2))
    Bexp = jnp.concatenate([sel.T, jnp.zeros((H, HF), f32)], axis=0)

    pad = E_PAD - E
    srcp = jnp.concatenate([edge_index[0], jnp.zeros((pad,), jnp.int32)])
    dstp = jnp.concatenate([edge_index[1], jnp.full((pad,), N, jnp.int32)])
    pk = (srcp + dstp * 16384).reshape(NW, CPT, CHUNK)

    proj, ss16, st16, c16 = pl.pallas_call(
        _k1_body,
        out_shape=[
            jax.ShapeDtypeStruct((N, HF), f32),
            jax.ShapeDtypeStruct((N, 2 * H), f32),
            jax.ShapeDtypeStruct((N, 2 * H), f32),
            jax.ShapeDtypeStruct((1, 2 * H), f32),
        ],
    )(x, W, A_src, A_tgt)
    c16 = c16.reshape(16)

    mesh = plsc.VectorSubcoreMesh(core_axis_name="c", subcore_axis_name="s")
    k2 = pl.kernel(
        _k2_body,
        out_type=(
            jax.ShapeDtypeStruct((NC, N, HF), f32),
            jax.ShapeDtypeStruct((NC, N, 2 * H), f32),
        ),
        mesh=mesh,
        compiler_params=pltpu.CompilerParams(use_tc_tiling_on_sc=False),
        scratch_types=[
            pltpu.VMEM((CHUNK,), jnp.int32),
            pltpu.VMEM((CHUNK,), jnp.int32),
            pltpu.VMEM((CHUNK,), jnp.int32),
            pltpu.VMEM((CHUNK, 2 * H), f32),
            pltpu.VMEM((CHUNK, 2 * H), f32),
            pltpu.VMEM((CHUNK, HF), f32),
            pltpu.VMEM((CHUNK, 2 * H), f32),
            pltpu.VMEM((16,), f32),
            pltpu.VMEM((16, HF), f32),
            pltpu.VMEM((16, 2 * H), f32),
            pltpu.VMEM_SHARED((NPAD, HF), f32),
            pltpu.VMEM_SHARED((NPAD, 2 * H), f32),
            pltpu.SemaphoreType.DMA,
        ],
    )
    acc2, den2 = k2(proj, ss16, st16, c16, pk)

    out = pl.pallas_call(
        _k3_body,
        out_shape=jax.ShapeDtypeStruct((N, HF), f32),
    )(acc2, den2, Bexp, bias)
    return out
